# baseline (speedup 1.0000x reference)
import functools

import jax
import jax.numpy as jnp
from jax import lax
from jax.experimental import pallas as pl
from jax.experimental.pallas import tpu as pltpu
from jax.experimental.pallas import tpu_sc as plsc

V, D, O = 1000000, 64, 256
B, L = 4096, 200

NC, NS = 2, 16
NW = NC * NS
EPW = B // NW
HALF = L // 2


def _make_sc_sum():
    mesh = plsc.VectorSubcoreMesh(core_axis_name="c", subcore_axis_name="s")

    @functools.partial(
        pl.kernel,
        mesh=mesh,
        compiler_params=pltpu.CompilerParams(use_tc_tiling_on_sc=False),
        out_type=jax.ShapeDtypeStruct((B, D), jnp.float32),
        scratch_types=[
            pltpu.VMEM((2 * EPW, HALF), jnp.int32),
            pltpu.VMEM((HALF, D), jnp.float32),
            pltpu.VMEM((HALF, D), jnp.float32),
            pltpu.VMEM((EPW, D), jnp.float32),
            pltpu.SemaphoreType.DMA,
        ],
    )
    def sc_sum(tok_hbm, table_hbm, out_hbm, tok_v, rows0, rows1, out_v, sem):
        wid = lax.axis_index("s") * NC + lax.axis_index("c")
        pltpu.sync_copy(tok_hbm.at[pl.ds(wid * (2 * EPW), 2 * EPW)], tok_v)

        def episode(e, carry):
            cp0 = pltpu.async_copy(table_hbm.at[tok_v.at[2 * e]], rows0, sem)
            cp1 = pltpu.async_copy(table_hbm.at[tok_v.at[2 * e + 1]], rows1, sem)
            cp0.wait()
            cp1.wait()

            z = jnp.zeros((16,), jnp.float32)

            def rbody(r, acc):
                a0, a1, a2, a3 = acc
                a0 = a0 + rows0[r, pl.ds(0, 16)] + rows1[r, pl.ds(0, 16)]
                a1 = a1 + rows0[r, pl.ds(16, 16)] + rows1[r, pl.ds(16, 16)]
                a2 = a2 + rows0[r, pl.ds(32, 16)] + rows1[r, pl.ds(32, 16)]
                a3 = a3 + rows0[r, pl.ds(48, 16)] + rows1[r, pl.ds(48, 16)]
                return (a0, a1, a2, a3)

            a0, a1, a2, a3 = lax.fori_loop(0, HALF, rbody, (z, z, z, z))
            out_v[e, pl.ds(0, 16)] = a0
            out_v[e, pl.ds(16, 16)] = a1
            out_v[e, pl.ds(32, 16)] = a2
            out_v[e, pl.ds(48, 16)] = a3
            return carry

        lax.fori_loop(0, EPW, episode, 0)
        pltpu.sync_copy(out_v, out_hbm.at[pl.ds(wid * EPW, EPW)])

    return sc_sum


def _mlp_body(tok_ref, sums_ref, w1_ref, b1_ref, w2_ref, b2_ref, out_ref):
    tok = tok_ref[...]
    cnt = jnp.sum((tok != 0).astype(jnp.float32), axis=1, keepdims=True)
    pooled = sums_ref[...] / jnp.maximum(cnt, 1.0)
    h = jnp.dot(pooled, w1_ref[...], precision=lax.Precision.HIGHEST,
                preferred_element_type=jnp.float32) + b1_ref[...]
    h = jnp.maximum(h, 0.0)
    p = jnp.dot(h, w2_ref[...], precision=lax.Precision.HIGHEST,
                preferred_element_type=jnp.float32) + b2_ref[...]
    nrm = jnp.sqrt(jnp.sum(p * p, axis=1, keepdims=True))
    out_ref[...] = p / jnp.maximum(nrm, 1e-8)


def _mlp(tokens, sums, W1, b1, W2, b2):
    blk = 1024
    grid = (B // blk,)
    return pl.pallas_call(
        _mlp_body,
        grid=grid,
        in_specs=[
            pl.BlockSpec((blk, L), lambda i: (i, 0)),
            pl.BlockSpec((blk, D), lambda i: (i, 0)),
            pl.BlockSpec((D, O), lambda i: (0, 0)),
            pl.BlockSpec((1, O), lambda i: (0, 0)),
            pl.BlockSpec((O, O), lambda i: (0, 0)),
            pl.BlockSpec((1, O), lambda i: (0, 0)),
        ],
        out_specs=pl.BlockSpec((blk, O), lambda i: (i, 0)),
        out_shape=jax.ShapeDtypeStruct((B, O), jnp.float32),
    )(tokens, sums, W1, b1, W2, b2)


def kernel(tokens, table, W1, b1, W2, b2):
    tok2 = tokens.reshape(2 * B, HALF)
    sums = _make_sc_sum()(tok2, table)
    return _mlp(tokens, sums, W1, b1.reshape(1, O), W2, b2.reshape(1, O))


# speedup vs baseline: 1.1308x; 1.1308x over previous
import functools

import jax
import jax.numpy as jnp
from jax import lax
from jax.experimental import pallas as pl
from jax.experimental.pallas import tpu as pltpu
from jax.experimental.pallas import tpu_sc as plsc

V, D, O = 1000000, 64, 256
B, L = 4096, 200

NC, NS = 2, 16
NW = NC * NS
EPW = B // NW
HALF = L // 2


def _make_sc_sum():
    mesh = plsc.VectorSubcoreMesh(core_axis_name="c", subcore_axis_name="s")

    @functools.partial(
        pl.kernel,
        mesh=mesh,
        compiler_params=pltpu.CompilerParams(use_tc_tiling_on_sc=False),
        out_type=jax.ShapeDtypeStruct((B, D), jnp.float32),
        scratch_types=[
            pltpu.VMEM((2 * EPW, HALF), jnp.int32),
            pltpu.VMEM((HALF, D), jnp.float32),
            pltpu.VMEM((HALF, D), jnp.float32),
            pltpu.VMEM((HALF, D), jnp.float32),
            pltpu.VMEM((HALF, D), jnp.float32),
            pltpu.VMEM((EPW, D), jnp.float32),
            pltpu.SemaphoreType.DMA,
            pltpu.SemaphoreType.DMA,
        ],
    )
    def sc_sum(tok_hbm, table_hbm, out_hbm, tok_v, a0_v, a1_v, b0_v, b1_v,
               out_v, sem_a, sem_b):
        wid = lax.axis_index("s") * NC + lax.axis_index("c")
        pltpu.sync_copy(tok_hbm.at[pl.ds(wid * (2 * EPW), 2 * EPW)], tok_v)

        def issue(e, r0, r1, sem):
            pltpu.async_copy(table_hbm.at[tok_v.at[2 * e]], r0, sem)
            pltpu.async_copy(table_hbm.at[tok_v.at[2 * e + 1]], r1, sem)

        def drain(r0, r1, sem):
            pltpu.make_async_copy(table_hbm.at[tok_v.at[0]], r0, sem).wait()
            pltpu.make_async_copy(table_hbm.at[tok_v.at[0]], r1, sem).wait()

        def sumbuf(e, r0, r1):
            z = jnp.zeros((16,), jnp.float32)

            def rbody(i, acc):
                a0, a1, a2, a3 = acc
                r = 2 * i
                a0 = a0 + r0[r, pl.ds(0, 16)] + r1[r, pl.ds(0, 16)]
                a1 = a1 + r0[r, pl.ds(16, 16)] + r1[r, pl.ds(16, 16)]
                a2 = a2 + r0[r, pl.ds(32, 16)] + r1[r, pl.ds(32, 16)]
                a3 = a3 + r0[r, pl.ds(48, 16)] + r1[r, pl.ds(48, 16)]
                s = r + 1
                a0 = a0 + r0[s, pl.ds(0, 16)] + r1[s, pl.ds(0, 16)]
                a1 = a1 + r0[s, pl.ds(16, 16)] + r1[s, pl.ds(16, 16)]
                a2 = a2 + r0[s, pl.ds(32, 16)] + r1[s, pl.ds(32, 16)]
                a3 = a3 + r0[s, pl.ds(48, 16)] + r1[s, pl.ds(48, 16)]
                return (a0, a1, a2, a3)

            a0, a1, a2, a3 = lax.fori_loop(0, HALF // 2, rbody, (z, z, z, z))
            out_v[e, pl.ds(0, 16)] = a0
            out_v[e, pl.ds(16, 16)] = a1
            out_v[e, pl.ds(32, 16)] = a2
            out_v[e, pl.ds(48, 16)] = a3

        issue(0, a0_v, a1_v, sem_a)

        def pair(i, carry):
            issue(2 * i + 1, b0_v, b1_v, sem_b)
            drain(a0_v, a1_v, sem_a)
            sumbuf(2 * i, a0_v, a1_v)

            @pl.when(i < EPW // 2 - 1)
            def _():
                issue(2 * i + 2, a0_v, a1_v, sem_a)

            drain(b0_v, b1_v, sem_b)
            sumbuf(2 * i + 1, b0_v, b1_v)
            return carry

        lax.fori_loop(0, EPW // 2, pair, 0)
        pltpu.sync_copy(out_v, out_hbm.at[pl.ds(wid * EPW, EPW)])

    return sc_sum


def _mlp_body(tok_ref, sums_ref, w1_ref, b1_ref, w2_ref, b2_ref, out_ref):
    tok = tok_ref[...]
    cnt = jnp.sum((tok != 0).astype(jnp.float32), axis=1, keepdims=True)
    pooled = sums_ref[...] / jnp.maximum(cnt, 1.0)
    h = jnp.dot(pooled, w1_ref[...], precision=lax.Precision.HIGHEST,
                preferred_element_type=jnp.float32) + b1_ref[...]
    h = jnp.maximum(h, 0.0)
    p = jnp.dot(h, w2_ref[...], precision=lax.Precision.HIGHEST,
                preferred_element_type=jnp.float32) + b2_ref[...]
    nrm = jnp.sqrt(jnp.sum(p * p, axis=1, keepdims=True))
    out_ref[...] = p / jnp.maximum(nrm, 1e-8)


def _mlp(tokens, sums, W1, b1, W2, b2):
    blk = 1024
    grid = (B // blk,)
    return pl.pallas_call(
        _mlp_body,
        grid=grid,
        in_specs=[
            pl.BlockSpec((blk, L), lambda i: (i, 0)),
            pl.BlockSpec((blk, D), lambda i: (i, 0)),
            pl.BlockSpec((D, O), lambda i: (0, 0)),
            pl.BlockSpec((1, O), lambda i: (0, 0)),
            pl.BlockSpec((O, O), lambda i: (0, 0)),
            pl.BlockSpec((1, O), lambda i: (0, 0)),
        ],
        out_specs=pl.BlockSpec((blk, O), lambda i: (i, 0)),
        out_shape=jax.ShapeDtypeStruct((B, O), jnp.float32),
    )(tokens, sums, W1, b1, W2, b2)


def kernel(tokens, table, W1, b1, W2, b2):
    tok2 = tokens.reshape(2 * B, HALF)
    sums = _make_sc_sum()(tok2, table)
    return _mlp(tokens, sums, W1, b1.reshape(1, O), W2, b2.reshape(1, O))


# speedup vs baseline: 1.2019x; 1.0629x over previous
import functools

import jax
import jax.numpy as jnp
from jax import lax
from jax.experimental import pallas as pl
from jax.experimental.pallas import tpu as pltpu
from jax.experimental.pallas import tpu_sc as plsc

V, D, O = 1000000, 64, 256
B, L = 4096, 200

NC, NS = 2, 16
NW = NC * NS
EPW = B // NW
HALF = L // 2


def _make_sc_sum():
    mesh = plsc.VectorSubcoreMesh(core_axis_name="c", subcore_axis_name="s")

    @functools.partial(
        pl.kernel,
        mesh=mesh,
        compiler_params=pltpu.CompilerParams(use_tc_tiling_on_sc=False),
        out_type=jax.ShapeDtypeStruct((B, D), jnp.float32),
        scratch_types=[
            pltpu.VMEM((2 * EPW, HALF), jnp.int32),
            pltpu.VMEM((HALF, D), jnp.float32),
            pltpu.VMEM((HALF, D), jnp.float32),
            pltpu.VMEM((HALF, D), jnp.float32),
            pltpu.VMEM((HALF, D), jnp.float32),
            pltpu.VMEM((EPW, D), jnp.float32),
            pltpu.SemaphoreType.DMA,
            pltpu.SemaphoreType.DMA,
        ],
    )
    def sc_sum(tok_hbm, table_hbm, out_hbm, tok_v, a0_v, a1_v, b0_v, b1_v,
               out_v, sem_a, sem_b):
        wid = lax.axis_index("s") * NC + lax.axis_index("c")
        pltpu.sync_copy(tok_hbm.at[pl.ds(wid * (2 * EPW), 2 * EPW)], tok_v)

        def issue(e, r0, r1, sem):
            pltpu.async_copy(table_hbm.at[tok_v.at[2 * e]], r0, sem)
            pltpu.async_copy(table_hbm.at[tok_v.at[2 * e + 1]], r1, sem)

        def drain(r0, r1, sem):
            pltpu.make_async_copy(table_hbm.at[tok_v.at[0]], r0, sem).wait()
            pltpu.make_async_copy(table_hbm.at[tok_v.at[0]], r1, sem).wait()

        def sumbuf(e, r0, r1):
            z = jnp.zeros((16,), jnp.float32)

            def rbody(i, acc):
                a0, a1, a2, a3 = acc
                r = 2 * i
                a0 = a0 + r0[r, pl.ds(0, 16)] + r1[r, pl.ds(0, 16)]
                a1 = a1 + r0[r, pl.ds(16, 16)] + r1[r, pl.ds(16, 16)]
                a2 = a2 + r0[r, pl.ds(32, 16)] + r1[r, pl.ds(32, 16)]
                a3 = a3 + r0[r, pl.ds(48, 16)] + r1[r, pl.ds(48, 16)]
                s = r + 1
                a0 = a0 + r0[s, pl.ds(0, 16)] + r1[s, pl.ds(0, 16)]
                a1 = a1 + r0[s, pl.ds(16, 16)] + r1[s, pl.ds(16, 16)]
                a2 = a2 + r0[s, pl.ds(32, 16)] + r1[s, pl.ds(32, 16)]
                a3 = a3 + r0[s, pl.ds(48, 16)] + r1[s, pl.ds(48, 16)]
                return (a0, a1, a2, a3)

            a0, a1, a2, a3 = lax.fori_loop(0, HALF // 2, rbody, (z, z, z, z))
            out_v[e, pl.ds(0, 16)] = a0
            out_v[e, pl.ds(16, 16)] = a1
            out_v[e, pl.ds(32, 16)] = a2
            out_v[e, pl.ds(48, 16)] = a3

        issue(0, a0_v, a1_v, sem_a)

        def pair(i, carry):
            issue(2 * i + 1, b0_v, b1_v, sem_b)
            drain(a0_v, a1_v, sem_a)
            sumbuf(2 * i, a0_v, a1_v)

            @pl.when(i < EPW // 2 - 1)
            def _():
                issue(2 * i + 2, a0_v, a1_v, sem_a)

            drain(b0_v, b1_v, sem_b)
            sumbuf(2 * i + 1, b0_v, b1_v)
            return carry

        lax.fori_loop(0, EPW // 2, pair, 0)
        pltpu.sync_copy(out_v, out_hbm.at[pl.ds(wid * EPW, EPW)])

    return sc_sum


PACK_C = 2048


def _pack_body(x_ref, o_ref):
    xt = jnp.swapaxes(x_ref[...], 0, 1)
    x3 = jnp.reshape(xt, (PACK_C // 2, 2, D))
    o_ref[:, 0:D] = x3[:, 0, :]
    o_ref[:, D:2 * D] = x3[:, 1, :]


def _pack_table(tableT):
    grid = (pl.cdiv(V, PACK_C),)
    return pl.pallas_call(
        _pack_body,
        grid=grid,
        in_specs=[pl.BlockSpec((D, PACK_C), lambda i: (0, i))],
        out_specs=pl.BlockSpec((PACK_C // 2, 2 * D), lambda i: (i, 0)),
        out_shape=jax.ShapeDtypeStruct((V // 2, 2 * D), jnp.float32),
    )(tableT)


def _mlp_body(tok_ref, sums_ref, w1_ref, b1_ref, w2_ref, b2_ref, out_ref):
    tok = tok_ref[...]
    cnt = jnp.sum((tok != 0).astype(jnp.float32), axis=1, keepdims=True)
    pooled = sums_ref[...] / jnp.maximum(cnt, 1.0)
    h = jnp.dot(pooled, w1_ref[...], precision=lax.Precision.HIGHEST,
                preferred_element_type=jnp.float32) + b1_ref[...]
    h = jnp.maximum(h, 0.0)
    p = jnp.dot(h, w2_ref[...], precision=lax.Precision.HIGHEST,
                preferred_element_type=jnp.float32) + b2_ref[...]
    nrm = jnp.sqrt(jnp.sum(p * p, axis=1, keepdims=True))
    out_ref[...] = p / jnp.maximum(nrm, 1e-8)


def _mlp(tokens, sums, W1, b1, W2, b2):
    blk = 1024
    grid = (B // blk,)
    return pl.pallas_call(
        _mlp_body,
        grid=grid,
        in_specs=[
            pl.BlockSpec((blk, L), lambda i: (i, 0)),
            pl.BlockSpec((blk, D), lambda i: (i, 0)),
            pl.BlockSpec((D, O), lambda i: (0, 0)),
            pl.BlockSpec((1, O), lambda i: (0, 0)),
            pl.BlockSpec((O, O), lambda i: (0, 0)),
            pl.BlockSpec((1, O), lambda i: (0, 0)),
        ],
        out_specs=pl.BlockSpec((blk, O), lambda i: (i, 0)),
        out_shape=jax.ShapeDtypeStruct((B, O), jnp.float32),
    )(tokens, sums, W1, b1, W2, b2)


def kernel(tokens, table, W1, b1, W2, b2):
    tok2 = tokens.reshape(2 * B, HALF)
    packed = _pack_table(table.T)
    lin = packed.reshape(V, D)
    sums = _make_sc_sum()(tok2, lin)
    return _mlp(tokens, sums, W1, b1.reshape(1, O), W2, b2.reshape(1, O))
